# Initial kernel scaffold; baseline (speedup 1.0000x reference)
import jax
import jax.numpy as jnp
from jax.experimental import pallas as pl


def kernel(output_tokens_from_bert):
    raise NotImplementedError("write your pallas kernel here")


# speedup vs baseline: 1.0129x; 1.0129x over previous
import jax
import jax.numpy as jnp
from jax import lax
from jax.experimental import pallas as pl
from jax.experimental.pallas import tpu as pltpu


SEQ = None

E_BLK = 256
S_BLK = 64

_ROT = ((13, 15, 26, 6), (17, 29, 16, 24))
_C240 = 0x1BD11BDA


def _threefry2x32(k0, k1, x0, x1):
    ks = (k0, k1, k0 ^ k1 ^ jnp.int32(_C240))
    x0 = x0 + ks[0]
    x1 = x1 + ks[1]
    for i in range(5):
        for r in _ROT[i % 2]:
            x0 = x0 + x1
            x1 = (x1 << r) | lax.shift_right_logical(x1, 32 - r)
            x1 = x1 ^ x0
        x0 = x0 + ks[(i + 1) % 3]
        x1 = x1 + ks[(i + 2) % 3] + jnp.int32(i + 1)
    return x0, x1


def _embrace_kernel(keys_ref, tokens_ref, out_ref):
    seq_len = tokens_ref.shape[1]
    emb = tokens_ref.shape[2]
    b = pl.program_id(0)
    k0 = keys_ref[b, 0]
    k1 = keys_ref[b, 1]

    n_schunk = seq_len // S_BLK

    sub = lax.broadcasted_iota(jnp.int32, (S_BLK, E_BLK), 0)
    lane = lax.broadcasted_iota(jnp.int32, (S_BLK, E_BLK), 1)
    sb_bits = S_BLK.bit_length() - 1
    revsub = (S_BLK - 1) - sub

    for ec in range(0, emb, E_BLK):
        f0 = (lane + ec) * seq_len + sub

        def s_body(si, carry, f0=f0):
            run_max, run_val = carry
            s0 = si * S_BLK
            f = f0 + s0
            o0, o1 = _threefry2x32(k0, k1, jnp.zeros_like(f), f)
            m = lax.shift_right_logical(o0 ^ o1, 9)
            packed = (m << sb_bits) | revsub
            c_pack = jnp.max(packed, axis=0, keepdims=True)
            cols = tokens_ref[0, pl.ds(s0, S_BLK), pl.ds(ec, E_BLK)]
            c_val = jnp.sum(
                jnp.where(packed == c_pack, cols, 0.0), axis=0, keepdims=True
            )
            c_max = lax.shift_right_logical(c_pack, sb_bits)
            upd = c_max > run_max
            return (
                jnp.where(upd, c_max, run_max),
                jnp.where(upd, c_val, run_val),
            )

        init = (
            jnp.full((1, E_BLK), -1, dtype=jnp.int32),
            jnp.zeros((1, E_BLK), dtype=jnp.float32),
        )
        _, val = lax.fori_loop(0, n_schunk, s_body, init)
        out_ref[0, :, pl.ds(ec, E_BLK)] = val


def kernel(output_tokens_from_bert):
    bs, seq_len, emb = output_tokens_from_bert.shape
    sample_key = jax.random.key(42)
    keys = jax.random.split(sample_key, bs)
    keys_i32 = lax.bitcast_convert_type(jax.random.key_data(keys), jnp.int32)

    out = pl.pallas_call(
        _embrace_kernel,
        grid=(bs,),
        in_specs=[
            pl.BlockSpec(memory_space=pltpu.SMEM),
            pl.BlockSpec((1, seq_len, emb), lambda b: (b, 0, 0)),
        ],
        out_specs=pl.BlockSpec((1, 1, emb), lambda b: (b, 0, 0)),
        out_shape=jax.ShapeDtypeStruct((bs, 1, emb), jnp.float32),
        compiler_params=pltpu.CompilerParams(
            dimension_semantics=("parallel",),
        ),
    )(keys_i32, output_tokens_from_bert)
    return out.reshape(bs, emb)


# speedup vs baseline: 1.0493x; 1.0359x over previous
import functools

import jax
import jax.numpy as jnp
from jax import lax
from jax.experimental import pallas as pl
from jax.experimental.pallas import tpu as pltpu
from jax.experimental.pallas import tpu_sc as plsc


E_BLK = 256
S_BLK = 64

_ROT = ((13, 15, 26, 6), (17, 29, 16, 24))
_C240 = 0x1BD11BDA


def _threefry2x32(k0, k1, x0, x1):
    ks = (k0, k1, k0 ^ k1 ^ jnp.int32(_C240))
    x0 = x0 + ks[0]
    x1 = x1 + ks[1]
    for i in range(5):
        for r in _ROT[i % 2]:
            x0 = x0 + x1
            x1 = (x1 << r) | lax.shift_right_logical(x1, 32 - r)
            x1 = x1 ^ x0
        x0 = x0 + ks[(i + 1) % 3]
        x1 = x1 + ks[(i + 2) % 3] + jnp.int32(i + 1)
    return x0, x1


def _sample_kernel(keys_ref, idx_ref, *, seq_len, emb):
    b = pl.program_id(0)
    k0 = keys_ref[b, 0]
    k1 = keys_ref[b, 1]

    n_schunk = seq_len // S_BLK

    sub = lax.broadcasted_iota(jnp.int32, (S_BLK, E_BLK), 0)
    lane = lax.broadcasted_iota(jnp.int32, (S_BLK, E_BLK), 1)
    sb_bits = S_BLK.bit_length() - 1
    revsub = (S_BLK - 1) - sub

    for ec in range(0, emb, E_BLK):
        f0 = (lane + ec) * seq_len + sub

        def s_body(si, carry, f0=f0):
            run_max, run_idx = carry
            s0 = si * S_BLK
            f = f0 + s0
            o0, o1 = _threefry2x32(k0, k1, jnp.zeros_like(f), f)
            m = lax.shift_right_logical(o0 ^ o1, 9)
            packed = (m << sb_bits) | revsub
            c_pack = jnp.max(packed, axis=0, keepdims=True)
            c_max = lax.shift_right_logical(c_pack, sb_bits)
            c_idx = (s0 + (S_BLK - 1)) - (c_pack & (S_BLK - 1))
            upd = c_max > run_max
            return (
                jnp.where(upd, c_max, run_max),
                jnp.where(upd, c_idx, run_idx),
            )

        init = (
            jnp.full((1, E_BLK), -1, dtype=jnp.int32),
            jnp.zeros((1, E_BLK), dtype=jnp.int32),
        )
        _, s_star = lax.fori_loop(0, n_schunk, s_body, init)
        lane_row = lax.broadcasted_iota(jnp.int32, (1, E_BLK), 1) + ec
        idx_ref[0, :, pl.ds(ec, E_BLK)] = (
            (b * seq_len + s_star) * emb + lane_row
        )


def _make_sc_gather(n_idx, per_w, n_workers, num_cores):
    mesh = plsc.VectorSubcoreMesh(core_axis_name="c", subcore_axis_name="s")

    @functools.partial(
        pl.kernel,
        mesh=mesh,
        out_type=jax.ShapeDtypeStruct((n_idx,), jnp.float32),
        scratch_types=[
            pltpu.VMEM((per_w,), jnp.int32),
            pltpu.VMEM((per_w,), jnp.float32),
            pltpu.SemaphoreType.DMA,
        ],
    )
    def sc_gather(tokens_hbm, idx_hbm, out_hbm, idx_v, vals_v, sem):
        wid = lax.axis_index("s") * num_cores + lax.axis_index("c")
        base = wid * per_w
        pltpu.sync_copy(idx_hbm.at[pl.ds(base, per_w)], idx_v)
        pltpu.async_copy(tokens_hbm.at[idx_v], vals_v, sem).wait()
        pltpu.sync_copy(vals_v, out_hbm.at[pl.ds(base, per_w)])

    return sc_gather


def kernel(output_tokens_from_bert):
    bs, seq_len, emb = output_tokens_from_bert.shape
    sample_key = jax.random.key(42)
    keys = jax.random.split(sample_key, bs)
    keys_i32 = lax.bitcast_convert_type(jax.random.key_data(keys), jnp.int32)

    flat_idx = pl.pallas_call(
        functools.partial(_sample_kernel, seq_len=seq_len, emb=emb),
        grid=(bs,),
        in_specs=[pl.BlockSpec(memory_space=pltpu.SMEM)],
        out_specs=pl.BlockSpec((1, 1, emb), lambda b: (b, 0, 0)),
        out_shape=jax.ShapeDtypeStruct((bs, 1, emb), jnp.int32),
        compiler_params=pltpu.CompilerParams(
            dimension_semantics=("parallel",),
        ),
    )(keys_i32)

    info = plsc.get_sparse_core_info()
    n_workers = info.num_cores * info.num_subcores
    n_idx = bs * emb
    per_w = n_idx // n_workers
    tokens_flat = output_tokens_from_bert.reshape(bs * seq_len * emb)
    vals = _make_sc_gather(n_idx, per_w, n_workers, info.num_cores)(
        tokens_flat, flat_idx.reshape(n_idx)
    )
    return vals.reshape(bs, emb)


# speedup vs baseline: 1.0577x; 1.0080x over previous
import functools

import jax
import jax.numpy as jnp
from jax import lax
from jax.experimental import pallas as pl
from jax.experimental.pallas import tpu as pltpu
from jax.experimental.pallas import tpu_sc as plsc


E_BLK = 256
S_BLK = 64

_ROT = ((13, 15, 26, 6), (17, 29, 16, 24))
_C240 = 0x1BD11BDA


def _threefry2x32(k0, k1, x0, x1):
    ks = (k0, k1, k0 ^ k1 ^ jnp.int32(_C240))
    for i in range(5):
        for r in _ROT[i % 2]:
            x0 = x0 + x1
            x1 = (x1 << r) | lax.shift_right_logical(x1, 32 - r)
            x1 = x1 ^ x0
        x0 = x0 + ks[(i + 1) % 3]
        x1 = x1 + ks[(i + 2) % 3] + jnp.int32(i + 1)
    return x0, x1


def _sample_kernel(keys_ref, idx_ref, *, seq_len, emb):
    b = pl.program_id(0)
    k0 = keys_ref[b, 0]
    k1 = keys_ref[b, 1]

    n_schunk = seq_len // S_BLK

    sub = lax.broadcasted_iota(jnp.int32, (S_BLK, E_BLK), 0)
    lane = lax.broadcasted_iota(jnp.int32, (S_BLK, E_BLK), 1)
    sb_bits = S_BLK.bit_length() - 1
    revsub = (S_BLK - 1) - sub

    x0_init = jnp.full((S_BLK, E_BLK), k0, dtype=jnp.int32)

    for ec in range(0, emb, E_BLK):
        f0k = (lane + ec) * seq_len + sub + k1

        def one_chunk(s0, carry, f0k=f0k):
            run_max, run_idx = carry
            o0, o1 = _threefry2x32(k0, k1, x0_init, f0k + s0)
            m = lax.shift_right_logical(o0 ^ o1, 9)
            packed = (m << sb_bits) | revsub
            c_pack = jnp.max(packed, axis=0, keepdims=True)
            c_max = lax.shift_right_logical(c_pack, sb_bits)
            c_idx = (s0 + (S_BLK - 1)) - (c_pack & (S_BLK - 1))
            upd = c_max > run_max
            return (
                jnp.where(upd, c_max, run_max),
                jnp.where(upd, c_idx, run_idx),
            )

        def s_body(si, carry):
            s0 = si * (2 * S_BLK)
            return one_chunk(s0 + S_BLK, one_chunk(s0, carry))

        init = (
            jnp.full((1, E_BLK), -1, dtype=jnp.int32),
            jnp.zeros((1, E_BLK), dtype=jnp.int32),
        )
        _, s_star = lax.fori_loop(0, n_schunk // 2, s_body, init)
        lane_row = lax.broadcasted_iota(jnp.int32, (1, E_BLK), 1) + ec
        idx_ref[0, :, pl.ds(ec, E_BLK)] = (
            (b * seq_len + s_star) * emb + lane_row
        )


def _make_sc_gather(n_idx, per_w, n_workers, num_cores):
    mesh = plsc.VectorSubcoreMesh(core_axis_name="c", subcore_axis_name="s")

    @functools.partial(
        pl.kernel,
        mesh=mesh,
        out_type=jax.ShapeDtypeStruct((n_idx,), jnp.float32),
        scratch_types=[
            pltpu.VMEM((per_w,), jnp.int32),
            pltpu.VMEM((per_w,), jnp.float32),
            pltpu.SemaphoreType.DMA,
        ],
    )
    def sc_gather(tokens_hbm, idx_hbm, out_hbm, idx_v, vals_v, sem):
        wid = lax.axis_index("s") * num_cores + lax.axis_index("c")
        base = wid * per_w
        pltpu.sync_copy(idx_hbm.at[pl.ds(base, per_w)], idx_v)
        pltpu.async_copy(tokens_hbm.at[idx_v], vals_v, sem).wait()
        pltpu.sync_copy(vals_v, out_hbm.at[pl.ds(base, per_w)])

    return sc_gather


def kernel(output_tokens_from_bert):
    bs, seq_len, emb = output_tokens_from_bert.shape
    sample_key = jax.random.key(42)
    keys = jax.random.split(sample_key, bs)
    keys_i32 = lax.bitcast_convert_type(jax.random.key_data(keys), jnp.int32)

    flat_idx = pl.pallas_call(
        functools.partial(_sample_kernel, seq_len=seq_len, emb=emb),
        grid=(bs,),
        in_specs=[pl.BlockSpec(memory_space=pltpu.SMEM)],
        out_specs=pl.BlockSpec((1, 1, emb), lambda b: (b, 0, 0)),
        out_shape=jax.ShapeDtypeStruct((bs, 1, emb), jnp.int32),
        compiler_params=pltpu.CompilerParams(
            dimension_semantics=("parallel",),
        ),
    )(keys_i32)

    info = plsc.get_sparse_core_info()
    n_workers = info.num_cores * info.num_subcores
    n_idx = bs * emb
    per_w = n_idx // n_workers
    tokens_flat = output_tokens_from_bert.reshape(bs * seq_len * emb)
    vals = _make_sc_gather(n_idx, per_w, n_workers, info.num_cores)(
        tokens_flat, flat_idx.reshape(n_idx)
    )
    return vals.reshape(bs, emb)


# speedup vs baseline: 1.0577x; 1.0000x over previous
import functools

import jax
import jax.numpy as jnp
from jax import lax
from jax.experimental import pallas as pl
from jax.experimental.pallas import tpu as pltpu
from jax.experimental.pallas import tpu_sc as plsc


E_BLK = 256
S_BLK = 64

_ROT = ((13, 15, 26, 6), (17, 29, 16, 24))
_C240 = 0x1BD11BDA


def _threefry2x32(k0, k1, x0, x1):
    ks = (k0, k1, k0 ^ k1 ^ jnp.int32(_C240))
    for i in range(5):
        for r in _ROT[i % 2]:
            x0 = x0 + x1
            x1 = (x1 << r) + lax.shift_right_logical(x1, 32 - r)
            x1 = x1 ^ x0
        x0 = x0 + ks[(i + 1) % 3]
        x1 = x1 + ks[(i + 2) % 3] + jnp.int32(i + 1)
    return x0, x1


def _sample_kernel(keys_ref, idx_ref, *, seq_len, emb):
    b = pl.program_id(0)
    k0 = keys_ref[b, 0]
    k1 = keys_ref[b, 1]

    n_schunk = seq_len // S_BLK

    sub = lax.broadcasted_iota(jnp.int32, (S_BLK, E_BLK), 0)
    lane = lax.broadcasted_iota(jnp.int32, (S_BLK, E_BLK), 1)
    sb_bits = S_BLK.bit_length() - 1
    revsub = (S_BLK - 1) - sub

    x0_init = jnp.full((S_BLK, E_BLK), k0, dtype=jnp.int32)

    for ec in range(0, emb, E_BLK):
        f0k = (lane + ec) * seq_len + sub + k1

        def one_chunk(s0, carry, f0k=f0k):
            run_max, run_idx = carry
            o0, o1 = _threefry2x32(k0, k1, x0_init, f0k + s0)
            m = lax.shift_right_logical(o0 ^ o1, 9)
            packed = (m << sb_bits) + revsub
            c_pack = jnp.max(packed, axis=0, keepdims=True)
            c_max = lax.shift_right_logical(c_pack, sb_bits)
            c_idx = (s0 + (S_BLK - 1)) - (c_pack & (S_BLK - 1))
            upd = c_max > run_max
            return (
                jnp.where(upd, c_max, run_max),
                jnp.where(upd, c_idx, run_idx),
            )

        def s_body(si, carry):
            s0 = si * (2 * S_BLK)
            return one_chunk(s0 + S_BLK, one_chunk(s0, carry))

        init = (
            jnp.full((1, E_BLK), -1, dtype=jnp.int32),
            jnp.zeros((1, E_BLK), dtype=jnp.int32),
        )
        _, s_star = lax.fori_loop(0, n_schunk // 2, s_body, init)
        lane_row = lax.broadcasted_iota(jnp.int32, (1, E_BLK), 1) + ec
        idx_ref[0, :, pl.ds(ec, E_BLK)] = (
            (b * seq_len + s_star) * emb + lane_row
        )


def _make_sc_gather(n_idx, per_w, n_workers, num_cores):
    mesh = plsc.VectorSubcoreMesh(core_axis_name="c", subcore_axis_name="s")

    @functools.partial(
        pl.kernel,
        mesh=mesh,
        out_type=jax.ShapeDtypeStruct((n_idx,), jnp.float32),
        scratch_types=[
            pltpu.VMEM((per_w,), jnp.int32),
            pltpu.VMEM((per_w,), jnp.float32),
            pltpu.SemaphoreType.DMA,
        ],
    )
    def sc_gather(tokens_hbm, idx_hbm, out_hbm, idx_v, vals_v, sem):
        wid = lax.axis_index("s") * num_cores + lax.axis_index("c")
        base = wid * per_w
        pltpu.sync_copy(idx_hbm.at[pl.ds(base, per_w)], idx_v)
        pltpu.async_copy(tokens_hbm.at[idx_v], vals_v, sem).wait()
        pltpu.sync_copy(vals_v, out_hbm.at[pl.ds(base, per_w)])

    return sc_gather


def kernel(output_tokens_from_bert):
    bs, seq_len, emb = output_tokens_from_bert.shape
    sample_key = jax.random.key(42)
    keys = jax.random.split(sample_key, bs)
    keys_i32 = lax.bitcast_convert_type(jax.random.key_data(keys), jnp.int32)

    flat_idx = pl.pallas_call(
        functools.partial(_sample_kernel, seq_len=seq_len, emb=emb),
        grid=(bs,),
        in_specs=[pl.BlockSpec(memory_space=pltpu.SMEM)],
        out_specs=pl.BlockSpec((1, 1, emb), lambda b: (b, 0, 0)),
        out_shape=jax.ShapeDtypeStruct((bs, 1, emb), jnp.int32),
        compiler_params=pltpu.CompilerParams(
            dimension_semantics=("parallel",),
        ),
    )(keys_i32)

    info = plsc.get_sparse_core_info()
    n_workers = info.num_cores * info.num_subcores
    n_idx = bs * emb
    per_w = n_idx // n_workers
    tokens_flat = output_tokens_from_bert.reshape(bs * seq_len * emb)
    vals = _make_sc_gather(n_idx, per_w, n_workers, info.num_cores)(
        tokens_flat, flat_idx.reshape(n_idx)
    )
    return vals.reshape(bs, emb)
